# Initial kernel scaffold; baseline (speedup 1.0000x reference)
import jax
import jax.numpy as jnp
from jax.experimental import pallas as pl


def kernel(x, W_fe, b_fe, W_exp, b_exp):
    raise NotImplementedError("write your pallas kernel here")


# speedup vs baseline: 1.4786x; 1.4786x over previous
import functools

import jax
import jax.numpy as jnp
from jax.experimental import pallas as pl
from jax.experimental.pallas import tpu as pltpu


def _body(x_ref, wfe_ref, bfe_ref, wexp_ref, bexp_ref, out_ref, feat_ref):
    e = pl.program_id(1)

    @pl.when(e == 0)
    def _():
        acc = jnp.dot(x_ref[...], wfe_ref[...],
                      preferred_element_type=jnp.float32)
        acc = acc + bfe_ref[...]
        feat_ref[...] = jnp.maximum(acc, 0.0).astype(jnp.bfloat16)

    logits = jnp.dot(feat_ref[...], wexp_ref[0],
                     preferred_element_type=jnp.float32)
    logits = logits + bexp_ref[0]
    out_ref[0] = jnp.maximum(logits, 0.0) + jnp.log1p(jnp.exp(-jnp.abs(logits)))


@functools.partial(jax.jit, static_argnames=("bn",))
def _geer(x, W_fe, b_fe, W_exp, b_exp, bn=1024):
    n, d = x.shape
    e, _, c = W_exp.shape
    bn = min(bn, n)
    xb = x.astype(jnp.bfloat16)
    wfeb = W_fe.astype(jnp.bfloat16)
    wexpb = W_exp.astype(jnp.bfloat16)
    bfe2 = b_fe.reshape(1, d).astype(jnp.float32)
    bexp2 = b_exp.reshape(e, 1, c).astype(jnp.float32)

    grid = (n // bn, e)
    return pl.pallas_call(
        _body,
        grid=grid,
        in_specs=[
            pl.BlockSpec((bn, d), lambda i, j: (i, 0)),
            pl.BlockSpec((d, d), lambda i, j: (0, 0)),
            pl.BlockSpec((1, d), lambda i, j: (0, 0)),
            pl.BlockSpec((1, d, c), lambda i, j: (j, 0, 0)),
            pl.BlockSpec((1, 1, c), lambda i, j: (j, 0, 0)),
        ],
        out_specs=pl.BlockSpec((1, bn, c), lambda i, j: (j, i, 0)),
        out_shape=jax.ShapeDtypeStruct((e, n, c), jnp.float32),
        scratch_shapes=[pltpu.VMEM((bn, d), jnp.bfloat16)],
        compiler_params=pltpu.CompilerParams(
            dimension_semantics=("arbitrary", "arbitrary"),
        ),
    )(xb, wfeb, bfe2, wexpb, bexp2)


def kernel(x, W_fe, b_fe, W_exp, b_exp):
    return _geer(x, W_fe, b_fe, W_exp, b_exp)
